# baseline (speedup 1.0000x reference)
import functools

import jax
import jax.numpy as jnp
from jax import lax
from jax.experimental import pallas as pl
from jax.experimental.pallas import tpu as pltpu
from jax.experimental.pallas import tpu_sc as plsc

N = 10000
K = 32
FEAT = 128
NLAYER = 6
H = 8
D = 16
HD = H * D

NCORES = 2
NSUB = 16
NW = NCORES * NSUB
NPAD = 10240
PERW = NPAD // NW
C = 2
CK = C * K
TBLW = HD + 16
NCHUNK = PERW // C
TROWS = 10000

_BLK = 512


def _tc_input_body(f_ref, w0_ref, b0_ref, o_ref):
    h = jnp.dot(f_ref[...], w0_ref[...], preferred_element_type=jnp.float32)
    o_ref[...] = jnp.maximum(h + b0_ref[...], 0.0)


def _tc_input(fpad, w0, b0row):
    return pl.pallas_call(
        _tc_input_body,
        grid=(NPAD // _BLK,),
        in_specs=[
            pl.BlockSpec((_BLK, FEAT), lambda i: (i, 0)),
            pl.BlockSpec((FEAT, HD), lambda i: (0, 0)),
            pl.BlockSpec((1, HD), lambda i: (0, 0)),
        ],
        out_specs=pl.BlockSpec((_BLK, HD), lambda i: (i, 0)),
        out_shape=jax.ShapeDtypeStruct((NPAD, HD), jnp.float32),
    )(fpad, w0, b0row)


def _tc_tbl_body(x_ref, w_ref, av_ref, tbl_ref):
    h = jnp.dot(x_ref[...], w_ref[...], preferred_element_type=jnp.float32)
    fidx = lax.broadcasted_iota(jnp.int32, (HD, H), 0)
    hidx = lax.broadcasted_iota(jnp.int32, (HD, H), 1)
    seg = (fidx // D == hidx).astype(jnp.float32)
    adst = jnp.dot(h * av_ref[0:1, :], seg,
                   preferred_element_type=jnp.float32)
    asrc = jnp.dot(h * av_ref[1:2, :], seg,
                   preferred_element_type=jnp.float32)
    tbl_ref[...] = jnp.concatenate(
        [h, jnp.concatenate([adst, asrc], axis=1)], axis=1)


def _tc_tbl(x, w, av):
    return pl.pallas_call(
        _tc_tbl_body,
        grid=(NPAD // _BLK,),
        in_specs=[
            pl.BlockSpec((_BLK, HD), lambda i: (i, 0)),
            pl.BlockSpec((HD, HD), lambda i: (0, 0)),
            pl.BlockSpec((2, HD), lambda i: (0, 0)),
        ],
        out_specs=pl.BlockSpec((_BLK, TBLW), lambda i: (i, 0)),
        out_shape=jax.ShapeDtypeStruct((NPAD, TBLW), jnp.float32),
    )(x, w, av)


def _sc_body(tbl_hbm, nbf_hbm, out_hbm,
             idx2, own_v, rows2, out2, tbl_sh, semg, sems, semo, semi):
    wid = lax.axis_index("s") * NCORES + lax.axis_index("c")
    base = wid * PERW
    sid = lax.axis_index("s")
    nper = TROWS // NSUB
    pltpu.async_copy(tbl_hbm.at[pl.ds(sid * nper, nper)],
                     tbl_sh.at[pl.ds(sid * nper, nper)], sems).wait()
    plsc.subcore_barrier()

    def idx_desc(i, b):
        return pltpu.make_async_copy(
            nbf_hbm.at[pl.ds((base + i * C) * K, CK)], idx2.at[b], semi)

    def gather_desc(i, b):
        return pltpu.make_async_copy(
            tbl_sh.at[idx2.at[b]], rows2.at[b], semg)

    idx_desc(0, 0).start()
    idx_desc(1, 1).start()
    idx_desc(0, 0).wait()
    gather_desc(0, 0).start()
    lanes = lax.iota(jnp.int32, 16)

    def compute(i, b):
        rows = rows2.at[b]
        own_off = jnp.minimum(base + i * C, TROWS - C)
        pltpu.sync_copy(
            tbl_sh.at[pl.ds(own_off, C), pl.ds(HD, 16)], own_v)
        for c in range(C):
            rowb = c * K
            owna = own_v[c, :]
            p0s, p1s, svs = [], [], []
            for hh in range(H):
                cidx = jnp.full((16,), HD + hh, jnp.int32)
                ad0 = plsc.load_gather(rows, [rowb + lanes, cidx])
                ad1 = plsc.load_gather(rows, [rowb + 16 + lanes, cidx])
                asc = owna[H + hh]
                e0 = ad0 + asc
                e1 = ad1 + asc
                e0 = jnp.where(e0 >= 0.0, e0, 0.2 * e0)
                e1 = jnp.where(e1 >= 0.0, e1, 0.2 * e1)
                p0 = jnp.exp(e0)
                p1 = jnp.exp(e1)
                s = plsc.cumsum(p0 + p1)[15]
                p0s.append(p0)
                p1s.append(p1)
                svs.append(s)
            accs = [p0s[hh][0] * rows[rowb, pl.ds(hh * D, D)]
                    for hh in range(H)]
            for k in range(1, 16):
                for hh in range(H):
                    accs[hh] = accs[hh] + (
                        p0s[hh][k] * rows[rowb + k, pl.ds(hh * D, D)])
            for k in range(16):
                for hh in range(H):
                    accs[hh] = accs[hh] + (
                        p1s[hh][k] * rows[rowb + 16 + k, pl.ds(hh * D, D)])
            for hh in range(H):
                o = accs[hh] / svs[hh]
                o = jnp.where(o > 0.0, o, jnp.exp(o) - 1.0)
                out2[b, c, pl.ds(hh * D, D)] = o

    def out_desc(i, b):
        return pltpu.make_async_copy(
            out2.at[b], out_hbm.at[pl.ds(base + i * C, C)], semo)

    def step(i2, i, b):
        idx_desc(lax.rem(i + 1, NCHUNK), 1 - b).wait()
        gather_desc(i, b).wait()
        gather_desc(lax.rem(i + 1, NCHUNK), 1 - b).start()
        idx_desc(lax.rem(i + 2, NCHUNK), b).start()

        @pl.when(i2 >= 1)
        def _():
            out_desc(i - 2, b).wait()

        compute(i, b)
        out_desc(i, b).start()

    def pair(i2, _):
        step(i2, i2 * 2, 0)
        step(i2, i2 * 2 + 1, 1)
        return ()

    lax.fori_loop(0, NCHUNK // 2, pair, ())
    gather_desc(0, 0).wait()
    idx_desc(1, 1).wait()
    out_desc(NCHUNK - 2, 0).wait()
    out_desc(NCHUNK - 1, 1).wait()


_sc_layer = pl.kernel(
    _sc_body,
    out_type=jax.ShapeDtypeStruct((NPAD, HD), jnp.float32),
    mesh=plsc.VectorSubcoreMesh(
        core_axis_name="c", subcore_axis_name="s",
        num_cores=NCORES, num_subcores=NSUB),
    scratch_types=[
        pltpu.VMEM((2, CK), jnp.int32),
        pltpu.VMEM((C, 16), jnp.float32),
        pltpu.VMEM((2, CK, TBLW), jnp.float32),
        pltpu.VMEM((2, C, HD), jnp.float32),
        pltpu.MemorySpace.VMEM_SHARED((TROWS, TBLW), jnp.float32),
        pltpu.SemaphoreType.DMA,
        pltpu.SemaphoreType.DMA,
        pltpu.SemaphoreType.DMA,
        pltpu.SemaphoreType.DMA,
    ],
    compiler_params=pltpu.CompilerParams(
        use_tc_tiling_on_sc=False, needs_layout_passes=False),
)


def kernel(feature, nb_id, W0, b0, Ws, a_src, a_dst):
    fpad = jnp.pad(feature, ((0, NPAD - N), (0, 0)))
    nbf = jnp.pad(nb_id.astype(jnp.int32), ((0, NPAD - N), (0, 0))).reshape(-1)
    x = _tc_input(fpad, W0, b0.reshape(1, HD))
    for i in range(NLAYER):
        av = jnp.stack([a_dst[i].reshape(HD), a_src[i].reshape(HD)])
        tbl = _tc_tbl(x, Ws[i], av)
        x = _sc_layer(tbl, nbf)
    return x[:N]


# speedup vs baseline: 1.0870x; 1.0870x over previous
import functools

import jax
import jax.numpy as jnp
from jax import lax
from jax.experimental import pallas as pl
from jax.experimental.pallas import tpu as pltpu
from jax.experimental.pallas import tpu_sc as plsc

N = 10000
K = 32
FEAT = 128
NLAYER = 6
H = 8
D = 16
HD = H * D

NCORES = 2
NSUB = 16
NW = NCORES * NSUB
NPAD = 10240
PERW = NPAD // NW
C = 2
CK = C * K
TBLW = HD + 16
NCHUNK = PERW // C
TROWS = 10000

_BLK = 512


def _tc_input_body(f_ref, w0_ref, b0_ref, o_ref):
    h = jnp.dot(f_ref[...], w0_ref[...], preferred_element_type=jnp.float32)
    o_ref[...] = jnp.maximum(h + b0_ref[...], 0.0)


def _tc_input(fpad, w0, b0row):
    return pl.pallas_call(
        _tc_input_body,
        grid=(NPAD // _BLK,),
        in_specs=[
            pl.BlockSpec((_BLK, FEAT), lambda i: (i, 0)),
            pl.BlockSpec((FEAT, HD), lambda i: (0, 0)),
            pl.BlockSpec((1, HD), lambda i: (0, 0)),
        ],
        out_specs=pl.BlockSpec((_BLK, HD), lambda i: (i, 0)),
        out_shape=jax.ShapeDtypeStruct((NPAD, HD), jnp.float32),
    )(fpad, w0, b0row)


def _tc_tbl_body(x_ref, w_ref, av_ref, tbl_ref):
    h = jnp.dot(x_ref[...], w_ref[...], preferred_element_type=jnp.float32)
    fidx = lax.broadcasted_iota(jnp.int32, (HD, H), 0)
    hidx = lax.broadcasted_iota(jnp.int32, (HD, H), 1)
    seg = (fidx // D == hidx).astype(jnp.float32)
    adst = jnp.dot(h * av_ref[0:1, :], seg,
                   preferred_element_type=jnp.float32)
    asrc = jnp.dot(h * av_ref[1:2, :], seg,
                   preferred_element_type=jnp.float32)
    tbl_ref[...] = jnp.concatenate(
        [h, jnp.concatenate([adst, asrc], axis=1)], axis=1)


def _tc_tbl(x, w, av):
    return pl.pallas_call(
        _tc_tbl_body,
        grid=(NPAD // _BLK,),
        in_specs=[
            pl.BlockSpec((_BLK, HD), lambda i: (i, 0)),
            pl.BlockSpec((HD, HD), lambda i: (0, 0)),
            pl.BlockSpec((2, HD), lambda i: (0, 0)),
        ],
        out_specs=pl.BlockSpec((_BLK, TBLW), lambda i: (i, 0)),
        out_shape=jax.ShapeDtypeStruct((NPAD, TBLW), jnp.float32),
    )(x, w, av)


def _sc_body(tbl_hbm, nbf_hbm, out_hbm,
             idx_all, asrc_all, rows2, out2, tbl_sh, semg, sems, semo):
    wid = lax.axis_index("s") * NCORES + lax.axis_index("c")
    base = wid * PERW
    sid = lax.axis_index("s")
    nper = TROWS // NSUB
    pltpu.async_copy(tbl_hbm.at[pl.ds(sid * nper, nper)],
                     tbl_sh.at[pl.ds(sid * nper, nper)], sems).wait()
    pltpu.sync_copy(nbf_hbm.at[pl.ds(base * K, PERW * K)], idx_all)
    plsc.subcore_barrier()
    own_off = jnp.minimum(base, TROWS - PERW)
    own_delta = base - own_off
    pltpu.sync_copy(tbl_sh.at[pl.ds(own_off, PERW), pl.ds(HD, 16)],
                    asrc_all)

    def gather_desc(i, b):
        return pltpu.make_async_copy(
            tbl_sh.at[idx_all.at[pl.ds(i * CK, CK)]], rows2.at[b], semg)

    gather_desc(0, 0).start()
    lanes = lax.iota(jnp.int32, 16)

    def compute(i, b):
        rows = rows2.at[b]
        for c in range(C):
            rowb = c * K
            oidx = jnp.minimum(own_delta + i * C + c, PERW - 1)
            owna = asrc_all[oidx, :]
            p0s, p1s, svs = [], [], []
            for hh in range(H):
                cidx = jnp.full((16,), HD + hh, jnp.int32)
                ad0 = plsc.load_gather(rows, [rowb + lanes, cidx])
                ad1 = plsc.load_gather(rows, [rowb + 16 + lanes, cidx])
                asc = owna[H + hh]
                e0 = ad0 + asc
                e1 = ad1 + asc
                e0 = jnp.where(e0 >= 0.0, e0, 0.2 * e0)
                e1 = jnp.where(e1 >= 0.0, e1, 0.2 * e1)
                p0 = jnp.exp(e0)
                p1 = jnp.exp(e1)
                s = plsc.cumsum(p0 + p1)[15]
                p0s.append(p0)
                p1s.append(p1)
                svs.append(s)
            accs = [p0s[hh][0] * rows[rowb, pl.ds(hh * D, D)]
                    for hh in range(H)]
            for k in range(1, 16):
                for hh in range(H):
                    accs[hh] = accs[hh] + (
                        p0s[hh][k] * rows[rowb + k, pl.ds(hh * D, D)])
            for k in range(16):
                for hh in range(H):
                    accs[hh] = accs[hh] + (
                        p1s[hh][k] * rows[rowb + 16 + k, pl.ds(hh * D, D)])
            for hh in range(H):
                o = accs[hh] / svs[hh]
                o = jnp.where(o > 0.0, o, jnp.exp(o) - 1.0)
                out2[b, c, pl.ds(hh * D, D)] = o

    def out_desc(i, b):
        return pltpu.make_async_copy(
            out2.at[b], out_hbm.at[pl.ds(base + i * C, C)], semo)

    def step(i2, i, b):
        gather_desc(i, b).wait()
        gather_desc(lax.rem(i + 1, NCHUNK), 1 - b).start()

        @pl.when(i2 >= 1)
        def _():
            out_desc(i - 2, b).wait()

        compute(i, b)
        out_desc(i, b).start()

    def pair(i2, _):
        step(i2, i2 * 2, 0)
        step(i2, i2 * 2 + 1, 1)
        return ()

    lax.fori_loop(0, NCHUNK // 2, pair, ())
    gather_desc(0, 0).wait()
    out_desc(NCHUNK - 2, 0).wait()
    out_desc(NCHUNK - 1, 1).wait()


_sc_layer = pl.kernel(
    _sc_body,
    out_type=jax.ShapeDtypeStruct((NPAD, HD), jnp.float32),
    mesh=plsc.VectorSubcoreMesh(
        core_axis_name="c", subcore_axis_name="s",
        num_cores=NCORES, num_subcores=NSUB),
    scratch_types=[
        pltpu.VMEM((PERW * K,), jnp.int32),
        pltpu.VMEM((PERW, 16), jnp.float32),
        pltpu.VMEM((2, CK, TBLW), jnp.float32),
        pltpu.VMEM((2, C, HD), jnp.float32),
        pltpu.MemorySpace.VMEM_SHARED((TROWS, TBLW), jnp.float32),
        pltpu.SemaphoreType.DMA,
        pltpu.SemaphoreType.DMA,
        pltpu.SemaphoreType.DMA,
    ],
    compiler_params=pltpu.CompilerParams(
        use_tc_tiling_on_sc=False, needs_layout_passes=False),
)


def kernel(feature, nb_id, W0, b0, Ws, a_src, a_dst):
    fpad = jnp.pad(feature, ((0, NPAD - N), (0, 0)))
    nbf = jnp.pad(nb_id.astype(jnp.int32), ((0, NPAD - N), (0, 0))).reshape(-1)
    x = _tc_input(fpad, W0, b0.reshape(1, HD))
    for i in range(NLAYER):
        av = jnp.stack([a_dst[i].reshape(HD), a_src[i].reshape(HD)])
        tbl = _tc_tbl(x, Ws[i], av)
        x = _sc_layer(tbl, nbf)
    return x[:N]


# speedup vs baseline: 1.1052x; 1.0167x over previous
import functools

import jax
import jax.numpy as jnp
from jax import lax
from jax.experimental import pallas as pl
from jax.experimental.pallas import tpu as pltpu
from jax.experimental.pallas import tpu_sc as plsc

N = 10000
K = 32
FEAT = 128
NLAYER = 6
H = 8
D = 16
HD = H * D

NCORES = 2
NSUB = 16
NW = NCORES * NSUB
NPAD = 10240
PERW = NPAD // NW
C = 2
CK = C * K
TBLW = HD + 16
NCHUNK = PERW // C
TROWS = 10000

_BLK = 512


def _alpha_and_pack(h, av):
    fidx = lax.broadcasted_iota(jnp.int32, (HD, H), 0)
    hidx = lax.broadcasted_iota(jnp.int32, (HD, H), 1)
    seg = (fidx // D == hidx).astype(jnp.float32)
    adst = jnp.dot(h * av[0:1, :], seg, preferred_element_type=jnp.float32)
    asrc = jnp.dot(h * av[1:2, :], seg, preferred_element_type=jnp.float32)
    return jnp.concatenate(
        [h, jnp.concatenate([adst, asrc], axis=1)], axis=1)


def _tc_tbl0_body(f_ref, w0_ref, b0_ref, w_ref, av_ref, tbl_ref):
    x = jnp.maximum(
        jnp.dot(f_ref[...], w0_ref[...], preferred_element_type=jnp.float32)
        + b0_ref[...], 0.0)
    h = jnp.dot(x, w_ref[...], preferred_element_type=jnp.float32)
    tbl_ref[...] = _alpha_and_pack(h, av_ref[...])


def _tc_tbl0(fpad, w0, b0row, w, av):
    return pl.pallas_call(
        _tc_tbl0_body,
        grid=(NPAD // _BLK,),
        in_specs=[
            pl.BlockSpec((_BLK, FEAT), lambda i: (i, 0)),
            pl.BlockSpec((FEAT, HD), lambda i: (0, 0)),
            pl.BlockSpec((1, HD), lambda i: (0, 0)),
            pl.BlockSpec((HD, HD), lambda i: (0, 0)),
            pl.BlockSpec((2, HD), lambda i: (0, 0)),
        ],
        out_specs=pl.BlockSpec((_BLK, TBLW), lambda i: (i, 0)),
        out_shape=jax.ShapeDtypeStruct((NPAD, TBLW), jnp.float32),
    )(fpad, w0, b0row, w, av)


def _tc_tbl_body(x_ref, w_ref, av_ref, tbl_ref):
    h = jnp.dot(x_ref[...], w_ref[...], preferred_element_type=jnp.float32)
    tbl_ref[...] = _alpha_and_pack(h, av_ref[...])


def _tc_tbl(x, w, av):
    return pl.pallas_call(
        _tc_tbl_body,
        grid=(NPAD // _BLK,),
        in_specs=[
            pl.BlockSpec((_BLK, HD), lambda i: (i, 0)),
            pl.BlockSpec((HD, HD), lambda i: (0, 0)),
            pl.BlockSpec((2, HD), lambda i: (0, 0)),
        ],
        out_specs=pl.BlockSpec((_BLK, TBLW), lambda i: (i, 0)),
        out_shape=jax.ShapeDtypeStruct((NPAD, TBLW), jnp.float32),
    )(x, w, av)


def _sc_body(tbl_hbm, nbf_hbm, out_hbm,
             idx_all, asrc_all, rows2, out2, tbl_sh, semg, sems, semo):
    wid = lax.axis_index("s") * NCORES + lax.axis_index("c")
    base = wid * PERW
    sid = lax.axis_index("s")
    nper = TROWS // NSUB
    pltpu.async_copy(tbl_hbm.at[pl.ds(sid * nper, nper)],
                     tbl_sh.at[pl.ds(sid * nper, nper)], sems).wait()
    pltpu.sync_copy(nbf_hbm.at[pl.ds(base * K, PERW * K)], idx_all)
    plsc.subcore_barrier()
    own_off = jnp.minimum(base, TROWS - PERW)
    own_delta = base - own_off
    pltpu.sync_copy(tbl_sh.at[pl.ds(own_off, PERW), pl.ds(HD, 16)],
                    asrc_all)

    def gather_desc(i, b):
        return pltpu.make_async_copy(
            tbl_sh.at[idx_all.at[pl.ds(i * CK, CK)]], rows2.at[b], semg)

    gather_desc(0, 0).start()
    lanes = lax.iota(jnp.int32, 16)

    def compute(i, b):
        rows = rows2.at[b]
        for c in range(C):
            rowb = c * K
            oidx = jnp.minimum(own_delta + i * C + c, PERW - 1)
            owna = asrc_all[oidx, :]
            p0s, p1s, svs = [], [], []
            for hh in range(H):
                cidx = jnp.full((16,), HD + hh, jnp.int32)
                ad0 = plsc.load_gather(rows, [rowb + lanes, cidx])
                ad1 = plsc.load_gather(rows, [rowb + 16 + lanes, cidx])
                asc = owna[H + hh]
                e0 = ad0 + asc
                e1 = ad1 + asc
                e0 = jnp.where(e0 >= 0.0, e0, 0.2 * e0)
                e1 = jnp.where(e1 >= 0.0, e1, 0.2 * e1)
                p0 = jnp.exp(e0)
                p1 = jnp.exp(e1)
                s = plsc.cumsum(p0 + p1)[15]
                p0s.append(p0)
                p1s.append(p1)
                svs.append(s)
            accs = [p0s[hh][0] * rows[rowb, pl.ds(hh * D, D)]
                    for hh in range(H)]
            for k in range(1, 16):
                for hh in range(H):
                    accs[hh] = accs[hh] + (
                        p0s[hh][k] * rows[rowb + k, pl.ds(hh * D, D)])
            for k in range(16):
                for hh in range(H):
                    accs[hh] = accs[hh] + (
                        p1s[hh][k] * rows[rowb + 16 + k, pl.ds(hh * D, D)])
            for hh in range(H):
                o = accs[hh] / svs[hh]
                o = jnp.where(o > 0.0, o, jnp.exp(o) - 1.0)
                out2[b, c, pl.ds(hh * D, D)] = o

    def out_desc(i, b):
        return pltpu.make_async_copy(
            out2.at[b], out_hbm.at[pl.ds(base + i * C, C)], semo)

    def step(i2, i, b):
        gather_desc(i, b).wait()
        gather_desc(lax.rem(i + 1, NCHUNK), 1 - b).start()

        @pl.when(i2 >= 1)
        def _():
            out_desc(i - 2, b).wait()

        compute(i, b)
        out_desc(i, b).start()

    def pair(i2, _):
        step(i2, i2 * 2, 0)
        step(i2, i2 * 2 + 1, 1)
        return ()

    lax.fori_loop(0, NCHUNK // 2, pair, ())
    gather_desc(0, 0).wait()
    out_desc(NCHUNK - 2, 0).wait()
    out_desc(NCHUNK - 1, 1).wait()


_sc_layer = pl.kernel(
    _sc_body,
    out_type=jax.ShapeDtypeStruct((NPAD, HD), jnp.float32),
    mesh=plsc.VectorSubcoreMesh(
        core_axis_name="c", subcore_axis_name="s",
        num_cores=NCORES, num_subcores=NSUB),
    scratch_types=[
        pltpu.VMEM((PERW * K,), jnp.int32),
        pltpu.VMEM((PERW, 16), jnp.float32),
        pltpu.VMEM((2, CK, TBLW), jnp.float32),
        pltpu.VMEM((2, C, HD), jnp.float32),
        pltpu.MemorySpace.VMEM_SHARED((TROWS, TBLW), jnp.float32),
        pltpu.SemaphoreType.DMA,
        pltpu.SemaphoreType.DMA,
        pltpu.SemaphoreType.DMA,
    ],
    compiler_params=pltpu.CompilerParams(
        use_tc_tiling_on_sc=False, needs_layout_passes=False),
)


def kernel(feature, nb_id, W0, b0, Ws, a_src, a_dst):
    fpad = jnp.pad(feature, ((0, NPAD - N), (0, 0)))
    nbf = jnp.pad(nb_id.astype(jnp.int32), ((0, NPAD - N), (0, 0))).reshape(-1)
    avs = [jnp.stack([a_dst[i].reshape(HD), a_src[i].reshape(HD)])
           for i in range(NLAYER)]
    tbl = _tc_tbl0(fpad, W0, b0.reshape(1, HD), Ws[0], avs[0])
    x = _sc_layer(tbl, nbf)
    for i in range(1, NLAYER):
        tbl = _tc_tbl(x, Ws[i], avs[i])
        x = _sc_layer(tbl, nbf)
    return x[:N]


# speedup vs baseline: 1.1520x; 1.0423x over previous
import functools

import jax
import jax.numpy as jnp
from jax import lax
from jax.experimental import pallas as pl
from jax.experimental.pallas import tpu as pltpu
from jax.experimental.pallas import tpu_sc as plsc

N = 10000
K = 32
FEAT = 128
NLAYER = 6
H = 8
D = 16
HD = H * D

NCORES = 2
NSUB = 16
NW = NCORES * NSUB
NPAD = 10240
PERW = NPAD // NW
C = 2
CK = C * K
TBLW = HD + 16
NCHUNK = PERW // C
TROWS = 10000

_BLK = 1024


def _alpha_and_pack(h, av):
    fidx = lax.broadcasted_iota(jnp.int32, (HD, H), 0)
    hidx = lax.broadcasted_iota(jnp.int32, (HD, H), 1)
    seg = (fidx // D == hidx).astype(jnp.float32)
    adst = jnp.dot(h * av[0:1, :], seg, preferred_element_type=jnp.float32)
    asrc = jnp.dot(h * av[1:2, :], seg, preferred_element_type=jnp.float32)
    return jnp.concatenate(
        [h, jnp.concatenate([adst, asrc], axis=1)], axis=1)


def _tc_tbl0_body(f_ref, w0_ref, b0_ref, w_ref, av_ref, tbl_ref):
    x = jnp.maximum(
        jnp.dot(f_ref[...], w0_ref[...], preferred_element_type=jnp.float32)
        + b0_ref[...], 0.0)
    h = jnp.dot(x, w_ref[...], preferred_element_type=jnp.float32)
    tbl_ref[...] = _alpha_and_pack(h, av_ref[...])


def _tc_tbl0(fpad, w0, b0row, w, av):
    return pl.pallas_call(
        _tc_tbl0_body,
        grid=(NPAD // _BLK,),
        in_specs=[
            pl.BlockSpec((_BLK, FEAT), lambda i: (i, 0)),
            pl.BlockSpec((FEAT, HD), lambda i: (0, 0)),
            pl.BlockSpec((1, HD), lambda i: (0, 0)),
            pl.BlockSpec((HD, HD), lambda i: (0, 0)),
            pl.BlockSpec((2, HD), lambda i: (0, 0)),
        ],
        out_specs=pl.BlockSpec((_BLK, TBLW), lambda i: (i, 0)),
        out_shape=jax.ShapeDtypeStruct((NPAD, TBLW), jnp.float32),
    )(fpad, w0, b0row, w, av)


def _tc_tbl_body(x_ref, w_ref, av_ref, tbl_ref):
    h = jnp.dot(x_ref[...], w_ref[...], preferred_element_type=jnp.float32)
    tbl_ref[...] = _alpha_and_pack(h, av_ref[...])


def _tc_tbl(x, w, av):
    return pl.pallas_call(
        _tc_tbl_body,
        grid=(NPAD // _BLK,),
        in_specs=[
            pl.BlockSpec((_BLK, HD), lambda i: (i, 0)),
            pl.BlockSpec((HD, HD), lambda i: (0, 0)),
            pl.BlockSpec((2, HD), lambda i: (0, 0)),
        ],
        out_specs=pl.BlockSpec((_BLK, TBLW), lambda i: (i, 0)),
        out_shape=jax.ShapeDtypeStruct((NPAD, TBLW), jnp.float32),
    )(x, w, av)


def _sc_body(tbl_hbm, nbf_hbm, out_hbm,
             idx_all, asrc_all, rows2, out2, tbl_sh, semg, sems, semo):
    wid = lax.axis_index("s") * NCORES + lax.axis_index("c")
    base = wid * PERW
    sid = lax.axis_index("s")
    nper = TROWS // NSUB
    pltpu.async_copy(tbl_hbm.at[pl.ds(sid * nper, nper)],
                     tbl_sh.at[pl.ds(sid * nper, nper)], sems).wait()
    pltpu.sync_copy(nbf_hbm.at[pl.ds(base * K, PERW * K)], idx_all)
    plsc.subcore_barrier()
    own_off = jnp.minimum(base, TROWS - PERW)
    own_delta = base - own_off
    pltpu.sync_copy(tbl_sh.at[pl.ds(own_off, PERW), pl.ds(HD, 16)],
                    asrc_all)

    def gather_desc(i, b):
        return pltpu.make_async_copy(
            tbl_sh.at[idx_all.at[pl.ds(i * CK, CK)]], rows2.at[b], semg)

    gather_desc(0, 0).start()
    lanes = lax.iota(jnp.int32, 16)

    def compute(i, b):
        rows = rows2.at[b]
        for c in range(C):
            rowb = c * K
            oidx = jnp.minimum(own_delta + i * C + c, PERW - 1)
            owna = asrc_all[oidx, :]
            p0s, p1s, svs = [], [], []
            for hh in range(H):
                cidx = jnp.full((16,), HD + hh, jnp.int32)
                ad0 = plsc.load_gather(rows, [rowb + lanes, cidx])
                ad1 = plsc.load_gather(rows, [rowb + 16 + lanes, cidx])
                asc = owna[H + hh]
                e0 = ad0 + asc
                e1 = ad1 + asc
                e0 = jnp.where(e0 >= 0.0, e0, 0.2 * e0)
                e1 = jnp.where(e1 >= 0.0, e1, 0.2 * e1)
                p0 = jnp.exp(e0)
                p1 = jnp.exp(e1)
                s = plsc.cumsum(p0 + p1)[15]
                p0s.append(p0)
                p1s.append(p1)
                svs.append(s)
            accs = [p0s[hh][0] * rows[rowb, pl.ds(hh * D, D)]
                    for hh in range(H)]
            for k in range(1, 16):
                for hh in range(H):
                    accs[hh] = accs[hh] + (
                        p0s[hh][k] * rows[rowb + k, pl.ds(hh * D, D)])
            for k in range(16):
                for hh in range(H):
                    accs[hh] = accs[hh] + (
                        p1s[hh][k] * rows[rowb + 16 + k, pl.ds(hh * D, D)])
            for hh in range(H):
                o = accs[hh] / svs[hh]
                o = jnp.where(o > 0.0, o, jnp.exp(o) - 1.0)
                out2[b, c, pl.ds(hh * D, D)] = o

    def out_desc(i, b):
        return pltpu.make_async_copy(
            out2.at[b], out_hbm.at[pl.ds(base + i * C, C)], semo)

    def step(i2, i, b):
        gather_desc(i, b).wait()
        gather_desc(lax.rem(i + 1, NCHUNK), 1 - b).start()

        @pl.when(i2 >= 1)
        def _():
            out_desc(i - 2, b).wait()

        compute(i, b)
        out_desc(i, b).start()

    def pair(i2, _):
        step(i2, i2 * 2, 0)
        step(i2, i2 * 2 + 1, 1)
        return ()

    lax.fori_loop(0, NCHUNK // 2, pair, ())
    gather_desc(0, 0).wait()
    out_desc(NCHUNK - 2, 0).wait()
    out_desc(NCHUNK - 1, 1).wait()


_sc_layer = pl.kernel(
    _sc_body,
    out_type=jax.ShapeDtypeStruct((NPAD, HD), jnp.float32),
    mesh=plsc.VectorSubcoreMesh(
        core_axis_name="c", subcore_axis_name="s",
        num_cores=NCORES, num_subcores=NSUB),
    scratch_types=[
        pltpu.VMEM((PERW * K,), jnp.int32),
        pltpu.VMEM((PERW, 16), jnp.float32),
        pltpu.VMEM((2, CK, TBLW), jnp.float32),
        pltpu.VMEM((2, C, HD), jnp.float32),
        pltpu.MemorySpace.VMEM_SHARED((TROWS, TBLW), jnp.float32),
        pltpu.SemaphoreType.DMA,
        pltpu.SemaphoreType.DMA,
        pltpu.SemaphoreType.DMA,
    ],
    compiler_params=pltpu.CompilerParams(
        use_tc_tiling_on_sc=False, needs_layout_passes=False),
)


def kernel(feature, nb_id, W0, b0, Ws, a_src, a_dst):
    fpad = jnp.pad(feature, ((0, NPAD - N), (0, 0)))
    nbf = jnp.pad(nb_id.astype(jnp.int32), ((0, NPAD - N), (0, 0))).reshape(-1)
    avs = [jnp.stack([a_dst[i].reshape(HD), a_src[i].reshape(HD)])
           for i in range(NLAYER)]
    tbl = _tc_tbl0(fpad, W0, b0.reshape(1, HD), Ws[0], avs[0])
    x = _sc_layer(tbl, nbf)
    for i in range(1, NLAYER):
        tbl = _tc_tbl(x, Ws[i], avs[i])
        x = _sc_layer(tbl, nbf)
    return x[:N]


# speedup vs baseline: 1.1903x; 1.0333x over previous
import functools

import jax
import jax.numpy as jnp
from jax import lax
from jax.experimental import pallas as pl
from jax.experimental.pallas import tpu as pltpu
from jax.experimental.pallas import tpu_sc as plsc

N = 10000
K = 32
FEAT = 128
NLAYER = 6
H = 8
D = 16
HD = H * D

NCORES = 2
NSUB = 16
NW = NCORES * NSUB
NPAD = 10240
PERW = NPAD // NW
C = 2
CK = C * K
TBLW = HD + 16
NCHUNK = PERW // C
TROWS = 10000

_BLK = 1024


def _alpha_and_pack(h, av):
    fidx = lax.broadcasted_iota(jnp.int32, (HD, H), 0)
    hidx = lax.broadcasted_iota(jnp.int32, (HD, H), 1)
    seg = (fidx // D == hidx).astype(jnp.float32)
    adst = jnp.dot(h * av[0:1, :], seg, preferred_element_type=jnp.float32)
    asrc = jnp.dot(h * av[1:2, :], seg, preferred_element_type=jnp.float32)
    return jnp.concatenate([adst, asrc], axis=1)


_TBL_OUT = dict(
    out_specs=[
        pl.BlockSpec((_BLK, HD), lambda i: (i, 0)),
        pl.BlockSpec((_BLK, 16), lambda i: (i, 0)),
    ],
    out_shape=[
        jax.ShapeDtypeStruct((NPAD, HD), jnp.float32),
        jax.ShapeDtypeStruct((NPAD, 16), jnp.float32),
    ],
)


def _tc_tbl0_body(f_ref, w0_ref, b0_ref, w_ref, av_ref, h_ref, a_ref):
    x = jnp.maximum(
        jnp.dot(f_ref[...], w0_ref[...], preferred_element_type=jnp.float32)
        + b0_ref[...], 0.0)
    h = jnp.dot(x, w_ref[...], preferred_element_type=jnp.float32)
    h_ref[...] = h
    a_ref[...] = _alpha_and_pack(h, av_ref[...])


def _tc_tbl0(fpad, w0, b0row, w, av):
    return pl.pallas_call(
        _tc_tbl0_body,
        grid=(NPAD // _BLK,),
        in_specs=[
            pl.BlockSpec((_BLK, FEAT), lambda i: (i, 0)),
            pl.BlockSpec((FEAT, HD), lambda i: (0, 0)),
            pl.BlockSpec((1, HD), lambda i: (0, 0)),
            pl.BlockSpec((HD, HD), lambda i: (0, 0)),
            pl.BlockSpec((2, HD), lambda i: (0, 0)),
        ],
        **_TBL_OUT,
    )(fpad, w0, b0row, w, av)


def _tc_tbl_body(x_ref, w_ref, av_ref, h_ref, a_ref):
    h = jnp.dot(x_ref[...], w_ref[...], preferred_element_type=jnp.float32)
    h_ref[...] = h
    a_ref[...] = _alpha_and_pack(h, av_ref[...])


def _tc_tbl(x, w, av):
    return pl.pallas_call(
        _tc_tbl_body,
        grid=(NPAD // _BLK,),
        in_specs=[
            pl.BlockSpec((_BLK, HD), lambda i: (i, 0)),
            pl.BlockSpec((HD, HD), lambda i: (0, 0)),
            pl.BlockSpec((2, HD), lambda i: (0, 0)),
        ],
        **_TBL_OUT,
    )(x, w, av)


def _sc_body(tbl_hbm, alp_hbm, nbf_hbm, out_hbm,
             idx_all, asrc_all, rows2, rowsa2, out2, tblh_sh, tbla_sh,
             semg, sems, semo):
    wid = lax.axis_index("s") * NCORES + lax.axis_index("c")
    base = wid * PERW
    sid = lax.axis_index("s")
    nper = TROWS // NSUB
    pltpu.async_copy(tbl_hbm.at[pl.ds(sid * nper, nper)],
                     tblh_sh.at[pl.ds(sid * nper, nper)], sems).wait()
    pltpu.async_copy(alp_hbm.at[pl.ds(sid * nper, nper)],
                     tbla_sh.at[pl.ds(sid * nper, nper)], sems).wait()
    pltpu.sync_copy(nbf_hbm.at[pl.ds(base * K, PERW * K)], idx_all)
    pltpu.sync_copy(alp_hbm.at[pl.ds(base, PERW)], asrc_all)
    plsc.subcore_barrier()

    def gather_desc(i, b):
        return pltpu.make_async_copy(
            tblh_sh.at[idx_all.at[pl.ds(i * CK, CK)]], rows2.at[b], semg)

    def gather_a_desc(i, b):
        return pltpu.make_async_copy(
            tbla_sh.at[idx_all.at[pl.ds(i * CK, CK)]], rowsa2.at[b], semg)

    gather_desc(0, 0).start()
    gather_a_desc(0, 0).start()
    lanes = lax.iota(jnp.int32, 16)

    def compute(i, b):
        rows = rows2.at[b]
        rowsa = rowsa2.at[b]
        for c in range(C):
            rowb = c * K
            owna = asrc_all[i * C + c, :]
            p0s, p1s, svs = [], [], []
            for hh in range(H):
                cidx = jnp.full((16,), hh, jnp.int32)
                ad0 = plsc.load_gather(rowsa, [rowb + lanes, cidx])
                ad1 = plsc.load_gather(rowsa, [rowb + 16 + lanes, cidx])
                asc = owna[H + hh]
                e0 = ad0 + asc
                e1 = ad1 + asc
                e0 = jnp.where(e0 >= 0.0, e0, 0.2 * e0)
                e1 = jnp.where(e1 >= 0.0, e1, 0.2 * e1)
                p0 = jnp.exp(e0)
                p1 = jnp.exp(e1)
                s = plsc.cumsum(p0 + p1)[15]
                p0s.append(p0)
                p1s.append(p1)
                svs.append(s)
            accs = [p0s[hh][0] * rows[rowb, pl.ds(hh * D, D)]
                    for hh in range(H)]
            for k in range(1, 16):
                for hh in range(H):
                    accs[hh] = accs[hh] + (
                        p0s[hh][k] * rows[rowb + k, pl.ds(hh * D, D)])
            for k in range(16):
                for hh in range(H):
                    accs[hh] = accs[hh] + (
                        p1s[hh][k] * rows[rowb + 16 + k, pl.ds(hh * D, D)])
            for hh in range(H):
                o = accs[hh] / svs[hh]
                o = jnp.where(o > 0.0, o, jnp.exp(o) - 1.0)
                out2[b, c, pl.ds(hh * D, D)] = o

    def out_desc(i, b):
        return pltpu.make_async_copy(
            out2.at[b], out_hbm.at[pl.ds(base + i * C, C)], semo)

    def step(i2, i, b):
        gather_desc(i, b).wait()
        gather_a_desc(i, b).wait()
        nxt = lax.rem(i + 1, NCHUNK)
        gather_desc(nxt, 1 - b).start()
        gather_a_desc(nxt, 1 - b).start()

        @pl.when(i2 >= 1)
        def _():
            out_desc(i - 2, b).wait()

        compute(i, b)
        out_desc(i, b).start()

    def pair(i2, _):
        step(i2, i2 * 2, 0)
        step(i2, i2 * 2 + 1, 1)
        return ()

    lax.fori_loop(0, NCHUNK // 2, pair, ())
    gather_desc(0, 0).wait()
    gather_a_desc(0, 0).wait()
    out_desc(NCHUNK - 2, 0).wait()
    out_desc(NCHUNK - 1, 1).wait()


_sc_layer = pl.kernel(
    _sc_body,
    out_type=jax.ShapeDtypeStruct((NPAD, HD), jnp.float32),
    mesh=plsc.VectorSubcoreMesh(
        core_axis_name="c", subcore_axis_name="s",
        num_cores=NCORES, num_subcores=NSUB),
    scratch_types=[
        pltpu.VMEM((PERW * K,), jnp.int32),
        pltpu.VMEM((PERW, 16), jnp.float32),
        pltpu.VMEM((2, CK, HD), jnp.float32),
        pltpu.VMEM((2, CK, 16), jnp.float32),
        pltpu.VMEM((2, C, HD), jnp.float32),
        pltpu.MemorySpace.VMEM_SHARED((TROWS, HD), jnp.float32),
        pltpu.MemorySpace.VMEM_SHARED((TROWS, 16), jnp.float32),
        pltpu.SemaphoreType.DMA,
        pltpu.SemaphoreType.DMA,
        pltpu.SemaphoreType.DMA,
    ],
    compiler_params=pltpu.CompilerParams(
        use_tc_tiling_on_sc=False, needs_layout_passes=False),
)


def kernel(feature, nb_id, W0, b0, Ws, a_src, a_dst):
    fpad = jnp.pad(feature, ((0, NPAD - N), (0, 0)))
    nbf = jnp.pad(nb_id.astype(jnp.int32), ((0, NPAD - N), (0, 0))).reshape(-1)
    avs = [jnp.stack([a_dst[i].reshape(HD), a_src[i].reshape(HD)])
           for i in range(NLAYER)]
    tbl, alp = _tc_tbl0(fpad, W0, b0.reshape(1, HD), Ws[0], avs[0])
    x = _sc_layer(tbl, alp, nbf)
    for i in range(1, NLAYER):
        tbl, alp = _tc_tbl(x, Ws[i], avs[i])
        x = _sc_layer(tbl, alp, nbf)
    return x[:N]
